# Initial kernel scaffold; baseline (speedup 1.0000x reference)
import jax
import jax.numpy as jnp
from jax.experimental import pallas as pl


def kernel(features, coords, time_frac, edge_index, W_e1, b_e1, W_e2, b_e2, W_x1, b_x1, W_x2, W_h1, b_h1, W_h2, b_h2):
    raise NotImplementedError("write your pallas kernel here")


# speedup vs baseline: 7.3972x; 7.3972x over previous
import jax
import jax.numpy as jnp
from jax import lax
from jax.experimental import pallas as pl
from jax.experimental.pallas import tpu as pltpu
from jax.experimental.pallas import tpu_sc as plsc

N = 10000
E = 320000
D = 128
NC = 2
NS = 16
NW = NC * NS
CH = 128
NCK = E // CH
BASE_TRIP = NCK // NW
EXTRA = NCK - BASE_TRIP * NW
MAXTRIP = BASE_TRIP + 1
BE = 2560
BN = 2000
RPT = N // NS
QN = 5 * N
QPAD = 51200

_f32 = jnp.float32
_i32 = jnp.int32


def _tables_body(f_ref, tf_ref, w1a_ref, w1b_ref, wt_ref, be1_ref,
                 wh1a_ref, bh1_ref, ta_ref, tb_ref, fh_ref):
    f = f_ref[:]
    ta_ref[:] = (jnp.dot(f, w1a_ref[:], preferred_element_type=_f32)
                 + tf_ref[:] * wt_ref[:] + be1_ref[:])
    tb_ref[:] = jnp.dot(f, w1b_ref[:], preferred_element_type=_f32)
    fh_ref[:] = jnp.dot(f, wh1a_ref[:], preferred_element_type=_f32) + bh1_ref[:]


def _build_tables(features, tf, w1a, w1b, wt, be1, wh1a, bh1):
    row = lambda i: (i, 0)
    full = lambda i: (0, 0)
    return pl.pallas_call(
        _tables_body,
        grid=(N // BN,),
        in_specs=[
            pl.BlockSpec((BN, D), row),
            pl.BlockSpec((BN, 1), row),
            pl.BlockSpec((D, D), full),
            pl.BlockSpec((D, D), full),
            pl.BlockSpec((1, D), full),
            pl.BlockSpec((1, D), full),
            pl.BlockSpec((D, D), full),
            pl.BlockSpec((1, D), full),
        ],
        out_specs=[pl.BlockSpec((BN, D), row)] * 3,
        out_shape=[jax.ShapeDtypeStruct((N, D), _f32)] * 3,
    )(features, tf, w1a, w1b, wt, be1, wh1a, bh1)


def _gather_body(ta, tb, carr, srcv, dstv, gdo, gso, d2o, cso,
                 i_d0, i_d1, i_d2, i_d3, i_s0, i_s1, i_s2, i_s3,
                 gdb0, gdb1, gsb0, gsb1, carr_v, d2b0, d2b1, csb0, csb1,
                 *sems):
    idxd = [i_d0, i_d1, i_d2, i_d3]
    idxs = [i_s0, i_s1, i_s2, i_s3]
    gdb = [gdb0, gdb1]
    gsb = [gsb0, gsb1]
    d2b = [d2b0, d2b1]
    csb = [csb0, csb1]
    sid_d = sems[0:4]
    sid_s = sems[4:8]
    sg_d = sems[8:10]
    sg_s = sems[10:12]
    ss_gd = sems[12:14]
    ss_gs = sems[14:16]
    ss_d2 = sems[16:18]
    ss_cs = sems[18:20]

    wid = lax.axis_index("s") * NC + lax.axis_index("c")
    trip = BASE_TRIP + jnp.where(wid < EXTRA, 1, 0)
    pltpu.sync_copy(carr, carr_v)

    def fire_idx(t, si):
        off = pl.multiple_of((wid + NW * t) * CH, 8)
        pltpu.async_copy(dstv.at[pl.ds(off, CH)], idxd[si], sid_d[si])
        pltpu.async_copy(srcv.at[pl.ds(off, CH)], idxs[si], sid_s[si])

    def wait_idx(si):
        pltpu.make_async_copy(dstv.at[pl.ds(0, CH)], idxd[si], sid_d[si]).wait()
        pltpu.make_async_copy(srcv.at[pl.ds(0, CH)], idxs[si], sid_s[si]).wait()

    def fire_gather(si, sd):
        pltpu.async_copy(ta.at[idxd[si]], gdb[sd], sg_d[sd])
        pltpu.async_copy(tb.at[idxs[si]], gsb[sd], sg_s[sd])

    def wait_gather(sd):
        pltpu.make_async_copy(ta.at[idxd[0]], gdb[sd], sg_d[sd]).wait()
        pltpu.make_async_copy(tb.at[idxs[0]], gsb[sd], sg_s[sd]).wait()

    def fire_store_big(t, sd):
        off = pl.multiple_of((wid + NW * t) * CH, 8)
        pltpu.async_copy(gdb[sd], gdo.at[pl.ds(off, CH)], ss_gd[sd])
        pltpu.async_copy(gsb[sd], gso.at[pl.ds(off, CH)], ss_gs[sd])

    def fire_store_small(t, sd):
        c = wid + NW * t
        pltpu.async_copy(d2b[sd],
                         d2o.at[pl.ds(pl.multiple_of(c * D, 8), D)], ss_d2[sd])
        pltpu.async_copy(csb[sd],
                         cso.at[pl.ds(pl.multiple_of(c * 3 * D, 8), 3 * D)],
                         ss_cs[sd])

    def wait_store(sd):
        pltpu.make_async_copy(gdb[sd], gdo.at[pl.ds(0, CH)], ss_gd[sd]).wait()
        pltpu.make_async_copy(gsb[sd], gso.at[pl.ds(0, CH)], ss_gs[sd]).wait()
        pltpu.make_async_copy(d2b[sd], d2o.at[pl.ds(0, D)], ss_d2[sd]).wait()
        pltpu.make_async_copy(csb[sd], cso.at[pl.ds(0, 3 * D)], ss_cs[sd]).wait()

    def compute(si, sd):
        for g in range(CH // 16):
            idv = idxd[si][pl.ds(g * 16, 16)]
            isv = idxs[si][pl.ds(g * 16, 16)]
            ad = idv * 4
            asr = isv * 4
            cdx = plsc.load_gather(carr_v, [ad])
            cdy = plsc.load_gather(carr_v, [ad + 1])
            cdz = plsc.load_gather(carr_v, [ad + 2])
            csx = plsc.load_gather(carr_v, [asr])
            csy = plsc.load_gather(carr_v, [asr + 1])
            csz = plsc.load_gather(carr_v, [asr + 2])
            dx = cdx - csx
            dy = cdy - csy
            dz = cdz - csz
            d2b[sd][pl.ds(g * 16, 16)] = dx * dx + dy * dy + dz * dz
            csb[sd][pl.ds(g * 16, 16)] = csx
            csb[sd][pl.ds(D + g * 16, 16)] = csy
            csb[sd][pl.ds(2 * D + g * 16, 16)] = csz

    fire_idx(0, 0)
    fire_idx(1, 1)
    wait_idx(0)
    fire_gather(0, 0)

    def outer(i, _):
        for b in range(4):
            j = i * 4 + b

            @pl.when(j < trip)
            def _(j=j, b=b):
                @pl.when(j + 2 < trip)
                def _():
                    fire_idx(j + 2, (b + 2) % 4)

                @pl.when(j >= 1)
                def _():
                    wait_store((b + 1) % 2)

                @pl.when(j + 1 < trip)
                def _():
                    wait_idx((b + 1) % 4)
                    fire_gather((b + 1) % 4, (b + 1) % 2)

                wait_gather(b % 2)
                fire_store_big(j, b % 2)
                compute(b % 4, b % 2)
                fire_store_small(j, b % 2)
        return 0

    lax.fori_loop(0, MAXTRIP // 4 + 1, outer, 0)

    @pl.when(wid < EXTRA)
    def _():
        wait_store((MAXTRIP - 1) % 2)

    @pl.when(wid >= EXTRA)
    def _():
        wait_store((BASE_TRIP - 1) % 2)


def _sc_gather(ta, tb, carr, src, dst):
    mesh = plsc.VectorSubcoreMesh(core_axis_name="c", subcore_axis_name="s")
    fn = pl.kernel(
        _gather_body,
        out_type=[
            jax.ShapeDtypeStruct((E, D), _f32),
            jax.ShapeDtypeStruct((E, D), _f32),
            jax.ShapeDtypeStruct((E,), _f32),
            jax.ShapeDtypeStruct((3 * E,), _f32),
        ],
        mesh=mesh,
        compiler_params=pltpu.CompilerParams(needs_layout_passes=False),
        scratch_types=(
            [pltpu.VMEM((CH,), _i32)] * 8
            + [pltpu.VMEM((CH, D), _f32)] * 4
            + [pltpu.VMEM((4 * N,), _f32)]
            + [pltpu.VMEM((D,), _f32)] * 2
            + [pltpu.VMEM((3 * D,), _f32)] * 2
            + [pltpu.SemaphoreType.DMA] * 20
        ),
    )
    return fn(ta, tb, carr, src, dst)


def _mlp_body(gd_ref, gs_ref, d2_ref, cs_ref, wd_ref, we2_ref, be2_ref,
              wx1_ref, bx1_ref, wx2_ref, me_ref, qp_ref):
    d2blk = d2_ref[0]
    parts = [
        lax.dot_general(d2blk[s:s + 1, :], wd_ref[:],
                        (((0,), (0,)), ((), ())), preferred_element_type=_f32)
        for s in range(BE // D)
    ]
    u = gd_ref[:] + gs_ref[:] + jnp.concatenate(parts, axis=0)
    m1 = u * jax.nn.sigmoid(u)
    m = jnp.dot(m1, we2_ref[:], preferred_element_type=_f32) + be2_ref[:]
    m = m * jax.nn.sigmoid(m)
    p = jnp.dot(m, wx1_ref[:], preferred_element_type=_f32) + bx1_ref[:]
    p = p * jax.nn.sigmoid(p)
    phi = jnp.tanh(jnp.sum(p * wx2_ref[:], axis=1, keepdims=True))
    me_ref[:] = m
    php = jnp.reshape(phi, (BE // D, D))
    cs = cs_ref[0]
    qp_ref[0, :, 0:D] = php
    qp_ref[0, :, D:2 * D] = php * cs[:, 0:D]
    qp_ref[0, :, 2 * D:3 * D] = php * cs[:, D:2 * D]
    qp_ref[0, :, 3 * D:4 * D] = php * cs[:, 2 * D:3 * D]


def _edge_mlp(gd, gs, d2p3, csp3, wd, we2, be2, wx1, bx1, wx2row):
    row = lambda i: (i, 0)
    full = lambda i: (0, 0)
    blk3 = lambda i: (i, 0, 0)
    nck_b = BE // CH
    return pl.pallas_call(
        _mlp_body,
        grid=(E // BE,),
        in_specs=[
            pl.BlockSpec((BE, D), row),
            pl.BlockSpec((BE, D), row),
            pl.BlockSpec((1, nck_b, D), blk3),
            pl.BlockSpec((1, nck_b, 3 * D), blk3),
            pl.BlockSpec((1, D), full),
            pl.BlockSpec((D, D), full),
            pl.BlockSpec((1, D), full),
            pl.BlockSpec((D, D), full),
            pl.BlockSpec((1, D), full),
            pl.BlockSpec((1, D), full),
        ],
        out_specs=[
            pl.BlockSpec((BE, D), row),
            pl.BlockSpec((1, nck_b, 4 * D), blk3),
        ],
        out_shape=[
            jax.ShapeDtypeStruct((E, D), _f32),
            jax.ShapeDtypeStruct((E // BE, nck_b, 4 * D), _f32),
        ],
    )(gd, gs, d2p3, csp3, wd, we2, be2, wx1, bx1, wx2row)


def _scatter_body(me, qp, dstv, pm, pq,
                  i_d0, i_d1, i_d2, i_d3, meb0, meb1, qb0, qb1,
                  i50, i51, i52, i53, i54, i55, i56, i57, i58, i59,
                  ones_v, zq, acc_m, acc_q, *sems):
    idxd = [i_d0, i_d1, i_d2, i_d3]
    meb = [meb0, meb1]
    qb = [qb0, qb1]
    i5 = [[i50, i51, i52, i53, i54], [i55, i56, i57, i58, i59]]
    zbuf = meb0
    sid_d = sems[0:4]
    sme = sems[4:6]
    sq = sems[6:8]
    sscm = sems[8:10]
    sscq = sems[10:12]
    cid = lax.axis_index("c")
    sid = lax.axis_index("s")
    wid = sid * NC + cid
    trip = BASE_TRIP + jnp.where(wid < EXTRA, 1, 0)

    def zrow(r, _):
        for k in range(D // 16):
            zbuf[r, pl.ds(k * 16, 16)] = jnp.zeros((16,), _f32)
        return 0

    lax.fori_loop(0, CH, zrow, 0)

    def zq_fill(k, _):
        zq[pl.ds(k * 16, 16)] = jnp.zeros((16,), _f32)
        return 0

    lax.fori_loop(0, (QPAD // NS) // 16, zq_fill, 0)

    @pl.when(sid < NS - 1)
    def _():
        for t in range(5):
            pltpu.sync_copy(zbuf, acc_m.at[pl.ds(sid * 640 + t * CH, CH)])

    @pl.when(sid == NS - 1)
    def _():
        for t in range(3):
            pltpu.sync_copy(zbuf, acc_m.at[pl.ds(9600 + t * CH, CH)])
        pltpu.sync_copy(zbuf.at[pl.ds(0, 16)], acc_m.at[pl.ds(9984, 16)])

    pltpu.sync_copy(zq, acc_q.at[pl.ds(sid * (QPAD // NS), QPAD // NS)])
    for g in range(CH // 16):
        ones_v[pl.ds(g * 16, 16)] = jnp.ones((16,), _f32)
    plsc.subcore_barrier()

    def fire_idx(t, si):
        off = pl.multiple_of((wid + NW * t) * CH, 8)
        pltpu.async_copy(dstv.at[pl.ds(off, CH)], idxd[si], sid_d[si])

    def wait_idx(si):
        pltpu.make_async_copy(dstv.at[pl.ds(0, CH)], idxd[si], sid_d[si]).wait()

    def fire_loads(t, s):
        c = wid + NW * t
        off = pl.multiple_of(c * CH, 8)
        pltpu.async_copy(me.at[pl.ds(off, CH)], meb[s], sme[s])
        offq = pl.multiple_of(c * 4 * D, 8)
        pltpu.async_copy(qp.at[pl.ds(offq, 4 * D)], qb[s], sq[s])

    def wait_loads(s):
        pltpu.make_async_copy(me.at[pl.ds(0, CH)], meb[s], sme[s]).wait()
        pltpu.make_async_copy(qp.at[pl.ds(0, 4 * D)], qb[s], sq[s]).wait()

    def compute_i5(si, s):
        for g in range(CH // 16):
            dv = idxd[si][pl.ds(g * 16, 16)] * 5
            for k in range(5):
                i5[s][k][pl.ds(g * 16, 16)] = dv + k

    def fire_scatters(si, s):
        pltpu.async_copy(meb[s], acc_m.at[idxd[si]], sscm[s], add=True)
        for k in range(4):
            pltpu.async_copy(qb[s].at[pl.ds(k * D, D)], acc_q.at[i5[s][k]],
                             sscq[s], add=True)
        pltpu.async_copy(ones_v, acc_q.at[i5[s][4]], sscq[s], add=True)

    def wait_scatters(s):
        pltpu.make_async_copy(meb[s], acc_m.at[idxd[0]], sscm[s]).wait()
        for k in range(4):
            pltpu.make_async_copy(qb[s].at[pl.ds(k * D, D)],
                                  acc_q.at[i5[s][k]], sscq[s]).wait()
        pltpu.make_async_copy(ones_v, acc_q.at[i5[s][4]], sscq[s]).wait()

    fire_idx(0, 0)
    fire_idx(1, 1)
    wait_idx(0)
    fire_loads(0, 0)

    def outer(i, _):
        for b in range(4):
            j = i * 4 + b

            @pl.when(j < trip)
            def _(j=j, b=b):
                @pl.when(j >= 1)
                def _():
                    wait_scatters((b + 1) % 2)

                @pl.when(j + 2 < trip)
                def _():
                    fire_idx(j + 2, (b + 2) % 4)

                @pl.when(j + 1 < trip)
                def _():
                    wait_idx((b + 1) % 4)
                    fire_loads(j + 1, (b + 1) % 2)

                wait_loads(b % 2)
                compute_i5(b % 4, b % 2)
                fire_scatters(b % 4, b % 2)
        return 0

    lax.fori_loop(0, MAXTRIP // 4 + 1, outer, 0)

    @pl.when(wid < EXTRA)
    def _():
        wait_scatters((MAXTRIP - 1) % 2)

    @pl.when(wid >= EXTRA)
    def _():
        wait_scatters((BASE_TRIP - 1) % 2)

    plsc.subcore_barrier()
    @pl.when(sid < NS - 1)
    def _():
        offm = pl.multiple_of(cid * N + sid * 640, 8)
        pltpu.sync_copy(acc_m.at[pl.ds(pl.multiple_of(sid * 640, 8), 640)],
                        pm.at[pl.ds(offm, 640)])

    @pl.when(sid == NS - 1)
    def _():
        offm = pl.multiple_of(cid * N + 9600, 8)
        pltpu.sync_copy(acc_m.at[pl.ds(9600, 400)],
                        pm.at[pl.ds(offm, 400)])
    offp = pl.multiple_of(cid * QPAD + sid * (QPAD // NS), 8)
    pltpu.sync_copy(acc_q.at[pl.ds(sid * (QPAD // NS), QPAD // NS)],
                    pq.at[pl.ds(offp, QPAD // NS)])


def _sc_scatter(me, qp2, dst):
    mesh = plsc.VectorSubcoreMesh(core_axis_name="c", subcore_axis_name="s")
    fn = pl.kernel(
        _scatter_body,
        out_type=[
            jax.ShapeDtypeStruct((NC * N, D), _f32),
            jax.ShapeDtypeStruct((NC * QPAD,), _f32),
        ],
        mesh=mesh,
        compiler_params=pltpu.CompilerParams(needs_layout_passes=False),
        scratch_types=(
            [pltpu.VMEM((CH,), _i32)] * 4
            + [pltpu.VMEM((CH, D), _f32)] * 2
            + [pltpu.VMEM((4 * D,), _f32)] * 2
            + [pltpu.VMEM((CH,), _i32)] * 10
            + [pltpu.VMEM((CH,), _f32)]
            + [pltpu.VMEM((QPAD // NS,), _f32)]
            + [pltpu.VMEM_SHARED((N, D), _f32)]
            + [pltpu.VMEM_SHARED((QPAD,), _f32)]
            + [pltpu.SemaphoreType.DMA] * 12
        ),
    )
    return fn(me, qp2, dst)


def _final_body(pm0_ref, pm1_ref, qq_ref, f_ref, c_ref, fh_ref, wh1b_ref,
                wh2_ref, bh2_ref, h_ref, x_ref):
    aggm = pm0_ref[:] + pm1_ref[:]
    qq = qq_ref[:]
    c = c_ref[:]
    s1 = qq[:, 0:1]
    s2 = qq[:, 1:4]
    cnt = qq[:, 4:5]
    x_ref[:] = c + (c * s1 - s2) * (1.0 / (cnt + 1.0))
    hpre = fh_ref[:] + jnp.dot(aggm, wh1b_ref[:], preferred_element_type=_f32)
    hs = hpre * jax.nn.sigmoid(hpre)
    h_ref[:] = (f_ref[:] + jnp.dot(hs, wh2_ref[:], preferred_element_type=_f32)
                + bh2_ref[:])


def _finalize(pm0, pm1, qq, features, coords, fh, wh1b, wh2, bh2):
    row = lambda i: (i, 0)
    full = lambda i: (0, 0)
    return pl.pallas_call(
        _final_body,
        grid=(N // BN,),
        in_specs=[
            pl.BlockSpec((BN, D), row),
            pl.BlockSpec((BN, D), row),
            pl.BlockSpec((BN, 5), row),
            pl.BlockSpec((BN, D), row),
            pl.BlockSpec((BN, 3), row),
            pl.BlockSpec((BN, D), row),
            pl.BlockSpec((D, D), full),
            pl.BlockSpec((D, D), full),
            pl.BlockSpec((1, D), full),
        ],
        out_specs=[
            pl.BlockSpec((BN, D), row),
            pl.BlockSpec((BN, 3), row),
        ],
        out_shape=[
            jax.ShapeDtypeStruct((N, D), _f32),
            jax.ShapeDtypeStruct((N, 3), _f32),
        ],
    )(pm0, pm1, qq, features, coords, fh, wh1b, wh2, bh2)


def kernel(features, coords, time_frac, edge_index, W_e1, b_e1, W_e2, b_e2,
           W_x1, b_x1, W_x2, W_h1, b_h1, W_h2, b_h2):
    src = edge_index[0]
    dst = edge_index[1]
    carr = jnp.concatenate([coords, jnp.zeros((N, 1), _f32)], 1).reshape(-1)

    ta, tb, fh = _build_tables(
        features, time_frac.reshape(N, 1), W_e1[:D], W_e1[D:2 * D],
        W_e1[2 * D + 1].reshape(1, D), b_e1.reshape(1, D), W_h1[:D],
        b_h1.reshape(1, D))

    gd, gs, d2f, csf = _sc_gather(ta, tb, carr, src, dst)

    me, qp = _edge_mlp(
        gd, gs, d2f.reshape(E // BE, BE // CH, D),
        csf.reshape(E // BE, BE // CH, 3 * D), W_e1[2 * D].reshape(1, D),
        W_e2, b_e2.reshape(1, D), W_x1, b_x1.reshape(1, D),
        W_x2.reshape(1, D))

    pm, pq = _sc_scatter(me, qp.reshape(-1), dst)
    pm = pm.reshape(NC, N, D)

    qq = (pq[:QN] + pq[QPAD:QPAD + QN]).reshape(N, 5)
    hout, xout = _finalize(pm[0], pm[1], qq, features, coords, fh,
                           W_h1[D:], W_h2, b_h2.reshape(1, D))

    x = xout - jnp.mean(xout, axis=0, keepdims=True)
    return jnp.concatenate([hout, x], axis=1)
